# baseline (device time: 52554 ns/iter reference)
import jax
import jax.numpy as jnp
from jax import lax
from jax.experimental import pallas as pl
from jax.experimental.pallas import tpu as pltpu

N_DEV = 8
M = 1024
CH = M // N_DEV


def kernel(x, W1, W2):
    def body(x_ref, w1_ref, w2_ref, out_ref,
             own_ref, rs_send, rs_recv, ag_send,
             rs_send_sems, rs_recv_sems, ag_send_sems, ag_recv_sems):
        my = lax.axis_index("i")

        barrier_sem = pltpu.get_barrier_semaphore()
        for off in range(1, N_DEV):
            peer = lax.rem(my + off, N_DEV)
            pl.semaphore_signal(
                barrier_sem, inc=1,
                device_id=(peer,), device_id_type=pl.DeviceIdType.MESH,
            )
        pl.semaphore_wait(barrier_sem, N_DEV - 1)

        w1 = w1_ref[...].astype(jnp.bfloat16)
        w2 = w2_ref[...].astype(jnp.bfloat16)

        def partial_chunk(c):
            xb = x_ref[pl.ds(c * CH, CH), :].astype(jnp.bfloat16)
            h = jnp.dot(xb, w1, preferred_element_type=jnp.float32)
            h = jnp.maximum(h, 0.0).astype(jnp.bfloat16)
            return jnp.dot(h, w2, preferred_element_type=jnp.float32)

        rdmas = []
        acc = None
        for k in range(N_DEV - 1):
            c = lax.rem(my + 1 + k, N_DEV)
            rs_send[k] = partial_chunk(c).astype(jnp.bfloat16)
            rdma = pltpu.make_async_remote_copy(
                src_ref=rs_send.at[k],
                dst_ref=rs_recv.at[k],
                send_sem=rs_send_sems.at[k],
                recv_sem=rs_recv_sems.at[k],
                device_id=(c,),
                device_id_type=pl.DeviceIdType.MESH,
            )
            rdma.start()
            rdmas.append(rdma)
            if k >= 2:
                rdmas[k - 2].wait_recv()
                add = rs_recv[k - 2].astype(jnp.float32)
                acc = add if acc is None else acc + add

        own_ref[...] = partial_chunk(my) + acc
        for k in range(N_DEV - 3, N_DEV - 1):
            rdmas[k].wait_recv()
            own_ref[...] = own_ref[...] + rs_recv[k].astype(jnp.float32)

        ag_send[...] = own_ref[...].astype(jnp.bfloat16)
        ag_rdmas = []
        for k in range(N_DEV - 1):
            d = lax.rem(my + 1 + k, N_DEV)
            rdma = pltpu.make_async_remote_copy(
                src_ref=ag_send,
                dst_ref=out_ref.at[pl.ds(my * CH, CH), :],
                send_sem=ag_send_sems.at[k],
                recv_sem=ag_recv_sems.at[k],
                device_id=(d,),
                device_id_type=pl.DeviceIdType.MESH,
            )
            rdma.start()
            ag_rdmas.append(rdma)

        out_ref[pl.ds(my * CH, CH), :] = ag_send[...]

        for k in range(N_DEV - 1):
            ag_rdmas[k].wait_recv()
            rdmas[k].wait_send()
            ag_rdmas[k].wait_send()

    n_slots = N_DEV - 1
    return pl.pallas_call(
        body,
        out_shape=jax.ShapeDtypeStruct((M, M), jnp.bfloat16),
        in_specs=[
            pl.BlockSpec(memory_space=pltpu.VMEM),
            pl.BlockSpec(memory_space=pltpu.VMEM),
            pl.BlockSpec(memory_space=pltpu.VMEM),
        ],
        out_specs=pl.BlockSpec(memory_space=pltpu.VMEM),
        scratch_shapes=[
            pltpu.VMEM((CH, M), jnp.float32),
            pltpu.VMEM((n_slots, CH, M), jnp.bfloat16),
            pltpu.VMEM((n_slots, CH, M), jnp.bfloat16),
            pltpu.VMEM((CH, M), jnp.bfloat16),
            pltpu.SemaphoreType.DMA((n_slots,)),
            pltpu.SemaphoreType.DMA((n_slots,)),
            pltpu.SemaphoreType.DMA((n_slots,)),
            pltpu.SemaphoreType.DMA((n_slots,)),
        ],
        compiler_params=pltpu.CompilerParams(collective_id=0),
    )(x, W1, W2)


# device time: 49213 ns/iter; 1.0679x vs baseline; 1.0679x over previous
import jax
import jax.numpy as jnp
from jax import lax
from jax.experimental import pallas as pl
from jax.experimental.pallas import tpu as pltpu

N_DEV = 8
M = 1024
CH = M // N_DEV


def kernel(x, W1, W2):
    def body(x_ref, w1_ref, w2_ref, out_ref,
             rs_send, rs_recv, ag_send,
             rs_send_sems, rs_recv_sems, ag_send_sems, ag_recv_sems):
        my = lax.axis_index("i")

        barrier_sem = pltpu.get_barrier_semaphore()
        for off in range(1, N_DEV):
            peer = lax.rem(my + off, N_DEV)
            pl.semaphore_signal(
                barrier_sem, inc=1,
                device_id=(peer,), device_id_type=pl.DeviceIdType.MESH,
            )
        pl.semaphore_wait(barrier_sem, N_DEV - 1)

        w1 = w1_ref[...].astype(jnp.bfloat16)
        w2 = w2_ref[...].astype(jnp.bfloat16)

        def partial_chunk(c):
            xb = x_ref[pl.ds(c * CH, CH), :].astype(jnp.bfloat16)
            h = jnp.dot(xb, w1, preferred_element_type=jnp.float32)
            h = jnp.maximum(h, 0.0).astype(jnp.bfloat16)
            return jnp.dot(h, w2, preferred_element_type=jnp.float32)

        rdmas = []
        for k in range(N_DEV - 1):
            c = lax.rem(my + 1 + k, N_DEV)
            rs_send[k] = partial_chunk(c).astype(jnp.bfloat16)
            rdma = pltpu.make_async_remote_copy(
                src_ref=rs_send.at[k],
                dst_ref=rs_recv.at[k],
                send_sem=rs_send_sems.at[k],
                recv_sem=rs_recv_sems.at[k],
                device_id=(c,),
                device_id_type=pl.DeviceIdType.MESH,
            )
            rdma.start()
            rdmas.append(rdma)

        own = partial_chunk(my)
        for k in range(N_DEV - 1):
            rdmas[k].wait_recv()
        total = own
        for k in range(N_DEV - 1):
            total = total + rs_recv[k].astype(jnp.float32)

        ag_send[...] = total.astype(jnp.bfloat16)
        ag_rdmas = []
        for k in range(N_DEV - 1):
            d = lax.rem(my + 1 + k, N_DEV)
            rdma = pltpu.make_async_remote_copy(
                src_ref=ag_send,
                dst_ref=out_ref.at[pl.ds(my * CH, CH), :],
                send_sem=ag_send_sems.at[k],
                recv_sem=ag_recv_sems.at[k],
                device_id=(d,),
                device_id_type=pl.DeviceIdType.MESH,
            )
            rdma.start()
            ag_rdmas.append(rdma)

        out_ref[pl.ds(my * CH, CH), :] = ag_send[...]

        for k in range(N_DEV - 1):
            ag_rdmas[k].wait_recv()
            rdmas[k].wait_send()
            ag_rdmas[k].wait_send()

    n_slots = N_DEV - 1
    return pl.pallas_call(
        body,
        out_shape=jax.ShapeDtypeStruct((M, M), jnp.bfloat16),
        in_specs=[
            pl.BlockSpec(memory_space=pltpu.VMEM),
            pl.BlockSpec(memory_space=pltpu.VMEM),
            pl.BlockSpec(memory_space=pltpu.VMEM),
        ],
        out_specs=pl.BlockSpec(memory_space=pltpu.VMEM),
        scratch_shapes=[
            pltpu.VMEM((n_slots, CH, M), jnp.bfloat16),
            pltpu.VMEM((n_slots, CH, M), jnp.bfloat16),
            pltpu.VMEM((CH, M), jnp.bfloat16),
            pltpu.SemaphoreType.DMA((n_slots,)),
            pltpu.SemaphoreType.DMA((n_slots,)),
            pltpu.SemaphoreType.DMA((n_slots,)),
            pltpu.SemaphoreType.DMA((n_slots,)),
        ],
        compiler_params=pltpu.CompilerParams(collective_id=0),
    )(x, W1, W2)
